# baseline (speedup 1.0000x reference)
import jax
import jax.numpy as jnp
from jax.experimental import pallas as pl
from jax.experimental.pallas import tpu as pltpu

_B, _D1, _F = 4096, 256, 128
_C = 512
_K = _B // _C


def _stream_kernel(x0_hbm, x1_hbm, x2_hbm, o_hbm,
                   x0_v, x1_v, x2_v, o_v, in_sem, out_sem):
    def in_copies(k):
        r = k * _C
        return (
            pltpu.make_async_copy(x0_hbm.at[pl.ds(r, _C)], x0_v.at[pl.ds(r, _C)], in_sem.at[k, 0]),
            pltpu.make_async_copy(x1_hbm.at[pl.ds(r, _C)], x1_v.at[pl.ds(r, _C)], in_sem.at[k, 1]),
            pltpu.make_async_copy(x2_hbm.at[pl.ds(r, _C)], x2_v.at[pl.ds(r, _C)], in_sem.at[k, 2]),
        )

    def out_copy(k):
        r = k * _C
        return pltpu.make_async_copy(o_v.at[pl.ds(r, _C)], o_hbm.at[pl.ds(r, _C)], out_sem.at[k])

    for k in range(_K):
        for c in in_copies(k):
            c.start()

    for k in range(_K):
        for c in in_copies(k):
            c.wait()
        r = k * _C
        x0 = x0_v[pl.ds(r, _C), :]
        x1 = x1_v[pl.ds(r, _C), :]
        mx = jnp.max(x1, axis=1, keepdims=True)
        mn = jnp.min(x1, axis=1, keepdims=True)
        o_v[pl.ds(r, _C), :_D1] = jnp.maximum(x0 * mx, x0 * mn)
        o_v[pl.ds(r, _C), _D1:] = x2_v[pl.ds(r, _C), :]
        out_copy(k).start()

    for k in range(_K):
        out_copy(k).wait()


def kernel(x0, x1, x2):
    return pl.pallas_call(
        _stream_kernel,
        in_specs=[
            pl.BlockSpec(memory_space=pl.ANY),
            pl.BlockSpec(memory_space=pl.ANY),
            pl.BlockSpec(memory_space=pl.ANY),
        ],
        out_specs=pl.BlockSpec(memory_space=pl.ANY),
        out_shape=jax.ShapeDtypeStruct((_B, _D1 + _F), x0.dtype),
        scratch_shapes=[
            pltpu.VMEM((_B, _D1), jnp.float32),
            pltpu.VMEM((_B, _D1), jnp.float32),
            pltpu.VMEM((_B, _F), jnp.float32),
            pltpu.VMEM((_B, _D1 + _F), jnp.float32),
            pltpu.SemaphoreType.DMA((_K, 3)),
            pltpu.SemaphoreType.DMA((_K,)),
        ],
    )(x0, x1, x2)


# speedup vs baseline: 1.0081x; 1.0081x over previous
import jax
import jax.numpy as jnp
from jax.experimental import pallas as pl
from jax.experimental.pallas import tpu as pltpu

_B, _D1, _F = 4096, 256, 128
_C = 512
_K = _B // _C


def _stream_kernel(x0_hbm, x1_hbm, x2_hbm, o_hbm, x1_v, o_v, in_sem, out_sem):
    def in_copies(k):
        r = k * _C
        rows = pl.ds(r, _C)
        return (
            pltpu.make_async_copy(x1_hbm.at[rows], x1_v.at[rows], in_sem.at[k, 0]),
            pltpu.make_async_copy(x0_hbm.at[rows], o_v.at[rows, pl.ds(0, _D1)], in_sem.at[k, 1]),
            pltpu.make_async_copy(x2_hbm.at[rows], o_v.at[rows, pl.ds(_D1, _F)], in_sem.at[k, 2]),
        )

    def out_copy(k):
        rows = pl.ds(k * _C, _C)
        return pltpu.make_async_copy(o_v.at[rows], o_hbm.at[rows], out_sem.at[k])

    for k in range(_K):
        for c in in_copies(k):
            c.start()

    for k in range(_K):
        for c in in_copies(k):
            c.wait()
        rows = pl.ds(k * _C, _C)
        x0 = o_v[rows, :_D1]
        x1 = x1_v[rows, :]
        mx = jnp.max(x1, axis=1, keepdims=True)
        mn = jnp.min(x1, axis=1, keepdims=True)
        o_v[rows, :_D1] = jnp.maximum(x0 * mx, x0 * mn)
        out_copy(k).start()

    for k in range(_K):
        out_copy(k).wait()


def kernel(x0, x1, x2):
    return pl.pallas_call(
        _stream_kernel,
        in_specs=[
            pl.BlockSpec(memory_space=pl.ANY),
            pl.BlockSpec(memory_space=pl.ANY),
            pl.BlockSpec(memory_space=pl.ANY),
        ],
        out_specs=pl.BlockSpec(memory_space=pl.ANY),
        out_shape=jax.ShapeDtypeStruct((_B, _D1 + _F), x0.dtype),
        scratch_shapes=[
            pltpu.VMEM((_B, _D1), jnp.float32),
            pltpu.VMEM((_B, _D1 + _F), jnp.float32),
            pltpu.SemaphoreType.DMA((_K, 3)),
            pltpu.SemaphoreType.DMA((_K,)),
        ],
    )(x0, x1, x2)


# speedup vs baseline: 1.0269x; 1.0187x over previous
import jax
import jax.numpy as jnp
from jax.experimental import pallas as pl
from jax.experimental.pallas import tpu as pltpu

_B, _D1, _F = 4096, 256, 128
_C = 1024
_K = _B // _C


def _stream_kernel(x0_hbm, x1_hbm, x2_hbm, o_hbm, x1_v, o_v, in_sem, out_sem):
    def in_copies(k):
        r = k * _C
        rows = pl.ds(r, _C)
        return (
            pltpu.make_async_copy(x1_hbm.at[rows], x1_v.at[rows], in_sem.at[k, 0]),
            pltpu.make_async_copy(x0_hbm.at[rows], o_v.at[rows, pl.ds(0, _D1)], in_sem.at[k, 1]),
            pltpu.make_async_copy(x2_hbm.at[rows], o_v.at[rows, pl.ds(_D1, _F)], in_sem.at[k, 2]),
        )

    def out_copy(k):
        rows = pl.ds(k * _C, _C)
        return pltpu.make_async_copy(o_v.at[rows], o_hbm.at[rows], out_sem.at[k])

    for k in range(_K):
        for c in in_copies(k):
            c.start()

    for k in range(_K):
        for c in in_copies(k):
            c.wait()
        rows = pl.ds(k * _C, _C)
        x0 = o_v[rows, :_D1]
        x1 = x1_v[rows, :]
        mx = jnp.max(x1, axis=1, keepdims=True)
        mn = jnp.min(x1, axis=1, keepdims=True)
        o_v[rows, :_D1] = jnp.maximum(x0 * mx, x0 * mn)
        out_copy(k).start()

    for k in range(_K):
        out_copy(k).wait()


def kernel(x0, x1, x2):
    return pl.pallas_call(
        _stream_kernel,
        in_specs=[
            pl.BlockSpec(memory_space=pl.ANY),
            pl.BlockSpec(memory_space=pl.ANY),
            pl.BlockSpec(memory_space=pl.ANY),
        ],
        out_specs=pl.BlockSpec(memory_space=pl.ANY),
        out_shape=jax.ShapeDtypeStruct((_B, _D1 + _F), x0.dtype),
        scratch_shapes=[
            pltpu.VMEM((_B, _D1), jnp.float32),
            pltpu.VMEM((_B, _D1 + _F), jnp.float32),
            pltpu.SemaphoreType.DMA((_K, 3)),
            pltpu.SemaphoreType.DMA((_K,)),
        ],
    )(x0, x1, x2)


# speedup vs baseline: 1.0416x; 1.0143x over previous
import jax
import jax.numpy as jnp
from jax.experimental import pallas as pl
from jax.experimental.pallas import tpu as pltpu

_B, _D1, _F = 4096, 256, 128
_C = 2048
_K = _B // _C


def _stream_kernel(x0_hbm, x1_hbm, x2_hbm, o_hbm, x1_v, o_v, in_sem, out_sem):
    def in_copies(k):
        r = k * _C
        rows = pl.ds(r, _C)
        return (
            pltpu.make_async_copy(x1_hbm.at[rows], x1_v.at[rows], in_sem.at[k, 0]),
            pltpu.make_async_copy(x0_hbm.at[rows], o_v.at[rows, pl.ds(0, _D1)], in_sem.at[k, 1]),
            pltpu.make_async_copy(x2_hbm.at[rows], o_v.at[rows, pl.ds(_D1, _F)], in_sem.at[k, 2]),
        )

    def out_copy(k):
        rows = pl.ds(k * _C, _C)
        return pltpu.make_async_copy(o_v.at[rows], o_hbm.at[rows], out_sem.at[k])

    for k in range(_K):
        for c in in_copies(k):
            c.start()

    for k in range(_K):
        for c in in_copies(k):
            c.wait()
        rows = pl.ds(k * _C, _C)
        x0 = o_v[rows, :_D1]
        x1 = x1_v[rows, :]
        mx = jnp.max(x1, axis=1, keepdims=True)
        mn = jnp.min(x1, axis=1, keepdims=True)
        o_v[rows, :_D1] = jnp.maximum(x0 * mx, x0 * mn)
        out_copy(k).start()

    for k in range(_K):
        out_copy(k).wait()


def kernel(x0, x1, x2):
    return pl.pallas_call(
        _stream_kernel,
        in_specs=[
            pl.BlockSpec(memory_space=pl.ANY),
            pl.BlockSpec(memory_space=pl.ANY),
            pl.BlockSpec(memory_space=pl.ANY),
        ],
        out_specs=pl.BlockSpec(memory_space=pl.ANY),
        out_shape=jax.ShapeDtypeStruct((_B, _D1 + _F), x0.dtype),
        scratch_shapes=[
            pltpu.VMEM((_B, _D1), jnp.float32),
            pltpu.VMEM((_B, _D1 + _F), jnp.float32),
            pltpu.SemaphoreType.DMA((_K, 3)),
            pltpu.SemaphoreType.DMA((_K,)),
        ],
    )(x0, x1, x2)


# speedup vs baseline: 1.0502x; 1.0083x over previous
import jax
import jax.numpy as jnp
from jax.experimental import pallas as pl
from jax.experimental.pallas import tpu as pltpu

_BLK = 2048


def _pairwise_max_kernel(x0_ref, x1_ref, x2_ref, out_ref):
    x0 = x0_ref[...]
    x1 = x1_ref[...]
    mx = jnp.max(x1, axis=1, keepdims=True)
    mn = jnp.min(x1, axis=1, keepdims=True)
    d1 = x0.shape[1]
    out_ref[:, :d1] = jnp.maximum(x0 * mx, x0 * mn)
    out_ref[:, d1:] = x2_ref[...]


def kernel(x0, x1, x2):
    B, D1 = x0.shape
    F = x2.shape[1]
    return pl.pallas_call(
        _pairwise_max_kernel,
        grid=(B // _BLK,),
        in_specs=[
            pl.BlockSpec((_BLK, D1), lambda i: (i, 0)),
            pl.BlockSpec((_BLK, x1.shape[1]), lambda i: (i, 0)),
            pl.BlockSpec((_BLK, F), lambda i: (i, 0)),
        ],
        out_specs=pl.BlockSpec((_BLK, D1 + F), lambda i: (i, 0)),
        out_shape=jax.ShapeDtypeStruct((B, D1 + F), x0.dtype),
        compiler_params=pltpu.CompilerParams(
            dimension_semantics=("parallel",),
        ),
    )(x0, x1, x2)
